# baseline (speedup 1.0000x reference)
import functools

import jax
import jax.numpy as jnp
from jax import lax
from jax.experimental import pallas as pl
from jax.experimental.pallas import tpu as pltpu
from jax.experimental.pallas import tpu_sc as plsc

try:
    _INFO = plsc.get_sparse_core_info()
    _NC = _INFO.num_cores
    _NS = _INFO.num_subcores
    _LANES = _INFO.num_lanes
except Exception:
    _NC, _NS, _LANES = 2, 16, 16
_NW = _NC * _NS

_BATCH = 16384
_EMBED = 16
_NUSER = 1000000
_NMOVIE = 100000
_NGENRE = 1000
_BPW = _BATCH // _NW
_GRP = 128
_NGRP = _BPW // _GRP

_UTILES = -(-_NUSER // 128)
_MTILES = -(-_NMOVIE // 128)
_TCBLK = 2048
_UNBLK = 4
_MNBLK = 1


def _expand_body(in_ref, out_ref):
    out_ref[0, 0] = in_ref[0]


def _tc_expand(t3, nblk):
    grid = (2, nblk)
    return pl.pallas_call(
        _expand_body,
        grid=grid,
        in_specs=[pl.BlockSpec((1, 8, _TCBLK * 128), lambda h, i: (h, 0, i))],
        out_specs=pl.BlockSpec((1, 1, 8, _TCBLK * 128),
                               lambda h, i: (h, i, 0, 0)),
        out_shape=jax.ShapeDtypeStruct((2, nblk, 8, _TCBLK * 128),
                                       jnp.float32),
    )(t3)


_BLKW = _TCBLK * 128


def _granule_rows(idx16):
    return (lax.shift_right_logical(idx16, 18) * (8 * 16384)
            + lax.bitwise_and(lax.shift_right_logical(idx16, 4), 16383))


def _mg_body(midx, gidx, mr, gt, mg,
             midx_f, gidx_f, idx_m, mbuf, gtbl, mgv, sem):
    wid = lax.axis_index("s") * _NC + lax.axis_index("c")
    base = wid * _BPW

    pltpu.sync_copy(midx.at[pl.ds(base, _BPW)], midx_f)
    pltpu.sync_copy(gidx.at[pl.ds(base, _BPW)], gidx_f)
    pltpu.sync_copy(gt, gtbl)

    lane = lax.iota(jnp.int32, _LANES)

    def group(g, carry):
        goff = pl.multiple_of(g * _GRP, _GRP)

        def idx_vec(k, _):
            koff = pl.multiple_of(k * _LANES, _LANES)
            bpm = _granule_rows(midx_f[pl.ds(goff + koff, _LANES)])
            for d in range(_EMBED):
                cm = ((d // 8) * _MNBLK * 8 + (d % 8)) * 16384
                idx_m[d, pl.ds(koff, _LANES)] = bpm + cm
            return _

        lax.fori_loop(0, _GRP // _LANES, idx_vec, 0)

        copies = [pltpu.async_copy(mr.at[idx_m.at[d]], mbuf.at[d], sem)
                  for d in range(_EMBED)]
        for c in copies:
            c.wait()

        def chunk(c, _):
            off = pl.multiple_of(c * _LANES, _LANES)
            row = off + lane
            mlane = lax.bitwise_and(midx_f[pl.ds(goff + off, _LANES)], 15)
            gi = gidx_f[pl.ds(goff + off, _LANES)]
            for d in range(_EMBED):
                col = jnp.full((_LANES,), d, jnp.int32)
                m = plsc.load_gather(mbuf, [col, row, mlane])
                gv = plsc.load_gather(gtbl, [gi, col])
                mgv[d, pl.ds(goff + off, _LANES)] = m + gv
            return _

        lax.fori_loop(0, _GRP // _LANES, chunk, 0)
        return carry

    lax.fori_loop(0, _NGRP, group, 0)

    pltpu.sync_copy(mgv, mg.at[wid])


def _make_dot_body16():
    def body(uidx, ur, mg, out, uidx_f, idx_u, ubuf, mgb, out_v, sem):
        wid = lax.axis_index("s") * _NC + lax.axis_index("c")
        base = wid * _BPW

        pltpu.sync_copy(uidx.at[pl.ds(base, _BPW)], uidx_f)
        pltpu.sync_copy(mg.at[wid], mgb)

        lane = lax.iota(jnp.int32, _LANES)

        def group(g, carry):
            goff = pl.multiple_of(g * _GRP, _GRP)

            def idx_vec(k, _):
                koff = pl.multiple_of(k * _LANES, _LANES)
                bpu = _granule_rows(uidx_f[pl.ds(goff + koff, _LANES)])
                for d in range(_EMBED):
                    cu = ((d // 8) * _UNBLK * 8 + (d % 8)) * 16384
                    idx_u[d, pl.ds(koff, _LANES)] = bpu + cu
                return _

            lax.fori_loop(0, _GRP // _LANES, idx_vec, 0)

            copies = [pltpu.async_copy(ur.at[idx_u.at[d]], ubuf.at[d], sem)
                      for d in range(_EMBED)]
            for c in copies:
                c.wait()

            def chunk(c, _):
                off = pl.multiple_of(c * _LANES, _LANES)
                row = off + lane
                ulane = lax.bitwise_and(
                    uidx_f[pl.ds(goff + off, _LANES)], 15)
                acc = jnp.zeros((_LANES,), jnp.float32)
                for d in range(_EMBED):
                    col = jnp.full((_LANES,), d, jnp.int32)
                    u = plsc.load_gather(ubuf, [col, row, ulane])
                    acc = acc + u * mgb[d, pl.ds(goff + off, _LANES)]
                out_v[pl.ds(goff + off, _LANES)] = acc
                return _

            lax.fori_loop(0, _GRP // _LANES, chunk, 0)
            return carry

        lax.fori_loop(0, _NGRP, group, 0)

        pltpu.sync_copy(out_v, out.at[pl.ds(base, _BPW)])

    return body


def _sc_mesh():
    return plsc.VectorSubcoreMesh(core_axis_name="c", subcore_axis_name="s",
                                  num_cores=_NC, num_subcores=_NS)


_SC_PARAMS = pltpu.CompilerParams(
    needs_layout_passes=False, use_tc_tiling_on_sc=False)


@jax.jit
def _run(ui, mi, gi, ut3, mt3, gt):
    mp = _tc_expand(mt3, _MNBLK)
    mr = mp.reshape(2 * _MNBLK * 8 * 16384, _LANES)

    mg_fn = functools.partial(
        pl.kernel,
        mesh=_sc_mesh(),
        compiler_params=_SC_PARAMS,
        out_type=jax.ShapeDtypeStruct((_NW, _EMBED, _BPW), jnp.float32),
        scratch_types=[
            pltpu.VMEM((_BPW,), jnp.int32),
            pltpu.VMEM((_BPW,), jnp.int32),
            pltpu.VMEM((_EMBED, _GRP), jnp.int32),
            pltpu.VMEM((_EMBED, _GRP, _LANES), jnp.float32),
            pltpu.VMEM((_NGENRE, _EMBED), jnp.float32),
            pltpu.VMEM((_EMBED, _BPW), jnp.float32),
            pltpu.SemaphoreType.DMA,
        ],
    )(_mg_body)
    mg = mg_fn(mi, gi, mr, gt)

    up = _tc_expand(ut3, _UNBLK)
    ur = up.reshape(2 * _UNBLK * 8 * 16384, _LANES)

    dot_fn = functools.partial(
        pl.kernel,
        mesh=_sc_mesh(),
        compiler_params=_SC_PARAMS,
        out_type=jax.ShapeDtypeStruct((_BATCH,), jnp.float32),
        scratch_types=[
            pltpu.VMEM((_BPW,), jnp.int32),
            pltpu.VMEM((_EMBED, _GRP), jnp.int32),
            pltpu.VMEM((_EMBED, _GRP, _LANES), jnp.float32),
            pltpu.VMEM((_EMBED, _BPW), jnp.float32),
            pltpu.VMEM((_BPW,), jnp.float32),
            pltpu.SemaphoreType.DMA,
        ],
    )(_make_dot_body16())
    return dot_fn(ui, ur, mg)


def kernel(user_indices, movie_indices, genre_indices,
           user_table, movie_table, genre_table):
    ui = user_indices.astype(jnp.int32)
    mi = movie_indices.astype(jnp.int32)
    gi = genre_indices.astype(jnp.int32)
    ut3 = user_table.T.reshape(2, 8, _NUSER)
    mt3 = movie_table.T.reshape(2, 8, _NMOVIE)
    return _run(ui, mi, gi, ut3, mt3, genre_table)


# speedup vs baseline: 1.9232x; 1.9232x over previous
import functools

import jax
import jax.numpy as jnp
from jax import lax
from jax.experimental import pallas as pl
from jax.experimental.pallas import tpu as pltpu
from jax.experimental.pallas import tpu_sc as plsc

try:
    _INFO = plsc.get_sparse_core_info()
    _NC = _INFO.num_cores
    _NS = _INFO.num_subcores
    _LANES = _INFO.num_lanes
except Exception:
    _NC, _NS, _LANES = 2, 16, 16
_NW = _NC * _NS

_BATCH = 16384
_EMBED = 16
_NUSER = 1000000
_NMOVIE = 100000
_NGENRE = 1000
_BPW = _BATCH // _NW
_GRP = 128
_NGRP = _BPW // _GRP

_UTILES = -(-_NUSER // 128)
_MTILES = -(-_NMOVIE // 128)
_TCBLK = 2048


def _expand_body(in_ref, out_ref):
    x = in_ref[0]
    for k in range(_TCBLK):
        out_ref[0, k] = x[:, k * 128:(k + 1) * 128]


def _tc_expand(t3, ntiles):
    grid = (2, -(-ntiles // _TCBLK))
    return pl.pallas_call(
        _expand_body,
        grid=grid,
        in_specs=[pl.BlockSpec((1, 8, _TCBLK * 128), lambda h, i: (h, 0, i))],
        out_specs=pl.BlockSpec((1, _TCBLK, 8, 128), lambda h, i: (h, i, 0, 0)),
        out_shape=jax.ShapeDtypeStruct((2, ntiles, 8, 128), jnp.float32),
    )(t3)


def _granule_rows(idx16):
    return (lax.shift_right_logical(idx16, 7) * 64
            + lax.bitwise_and(lax.shift_right_logical(idx16, 4), 7))


def _mg_body(midx, gidx, mr, gt, mg,
             midx_f, gidx_f, idx_m, mbuf, gtbl, mgv, sem):
    wid = lax.axis_index("s") * _NC + lax.axis_index("c")
    base = wid * _BPW

    pltpu.sync_copy(midx.at[pl.ds(base, _BPW)], midx_f)
    pltpu.sync_copy(gidx.at[pl.ds(base, _BPW)], gidx_f)
    pltpu.sync_copy(gt, gtbl)

    lane = lax.iota(jnp.int32, _LANES)

    def group(g, carry):
        goff = pl.multiple_of(g * _GRP, _GRP)

        def idx_vec(k, _):
            koff = pl.multiple_of(k * _LANES, _LANES)
            bpm = _granule_rows(midx_f[pl.ds(goff + koff, _LANES)])
            for d in range(_EMBED):
                cm = (d // 8) * _MTILES * 64 + (d % 8) * 8
                idx_m[d, pl.ds(koff, _LANES)] = bpm + cm
            return _

        lax.fori_loop(0, _GRP // _LANES, idx_vec, 0)

        copies = [pltpu.async_copy(mr.at[idx_m.at[d]], mbuf.at[d], sem)
                  for d in range(_EMBED)]
        for c in copies:
            c.wait()

        def chunk(c, _):
            off = pl.multiple_of(c * _LANES, _LANES)
            row = off + lane
            mlane = lax.bitwise_and(midx_f[pl.ds(goff + off, _LANES)], 15)
            gi = gidx_f[pl.ds(goff + off, _LANES)]
            for d in range(_EMBED):
                col = jnp.full((_LANES,), d, jnp.int32)
                m = plsc.load_gather(mbuf, [col, row, mlane])
                gv = plsc.load_gather(gtbl, [gi, col])
                mgv[d, pl.ds(goff + off, _LANES)] = m + gv
            return _

        lax.fori_loop(0, _GRP // _LANES, chunk, 0)
        return carry

    lax.fori_loop(0, _NGRP, group, 0)

    pltpu.sync_copy(mgv, mg.at[wid])


def _make_dot_body16():
    def body(uidx, ur, mg, out, uidx_f, idx_u, ubuf, mgb, out_v, sem):
        wid = lax.axis_index("s") * _NC + lax.axis_index("c")
        base = wid * _BPW

        pltpu.sync_copy(uidx.at[pl.ds(base, _BPW)], uidx_f)
        pltpu.sync_copy(mg.at[wid], mgb)

        lane = lax.iota(jnp.int32, _LANES)

        def group(g, carry):
            goff = pl.multiple_of(g * _GRP, _GRP)

            def idx_vec(k, _):
                koff = pl.multiple_of(k * _LANES, _LANES)
                bpu = _granule_rows(uidx_f[pl.ds(goff + koff, _LANES)])
                for d in range(_EMBED):
                    cu = (d // 8) * _UTILES * 64 + (d % 8) * 8
                    idx_u[d, pl.ds(koff, _LANES)] = bpu + cu
                return _

            lax.fori_loop(0, _GRP // _LANES, idx_vec, 0)

            copies = [pltpu.async_copy(ur.at[idx_u.at[d]], ubuf.at[d], sem)
                      for d in range(_EMBED)]
            for c in copies:
                c.wait()

            def chunk(c, _):
                off = pl.multiple_of(c * _LANES, _LANES)
                row = off + lane
                ulane = lax.bitwise_and(
                    uidx_f[pl.ds(goff + off, _LANES)], 15)
                acc = jnp.zeros((_LANES,), jnp.float32)
                for d in range(_EMBED):
                    col = jnp.full((_LANES,), d, jnp.int32)
                    u = plsc.load_gather(ubuf, [col, row, ulane])
                    acc = acc + u * mgb[d, pl.ds(goff + off, _LANES)]
                out_v[pl.ds(goff + off, _LANES)] = acc
                return _

            lax.fori_loop(0, _GRP // _LANES, chunk, 0)
            return carry

        lax.fori_loop(0, _NGRP, group, 0)

        pltpu.sync_copy(out_v, out.at[pl.ds(base, _BPW)])

    return body


def _sc_mesh():
    return plsc.VectorSubcoreMesh(core_axis_name="c", subcore_axis_name="s",
                                  num_cores=_NC, num_subcores=_NS)


_SC_PARAMS = pltpu.CompilerParams(
    needs_layout_passes=False, use_tc_tiling_on_sc=False)


@jax.jit
def _run(ui, mi, gi, ut3, mt3, gt):
    mp = _tc_expand(mt3, _MTILES)
    mr = mp.reshape(2 * _MTILES * 64, _LANES)

    mg_fn = functools.partial(
        pl.kernel,
        mesh=_sc_mesh(),
        compiler_params=_SC_PARAMS,
        out_type=jax.ShapeDtypeStruct((_NW, _EMBED, _BPW), jnp.float32),
        scratch_types=[
            pltpu.VMEM((_BPW,), jnp.int32),
            pltpu.VMEM((_BPW,), jnp.int32),
            pltpu.VMEM((_EMBED, _GRP), jnp.int32),
            pltpu.VMEM((_EMBED, _GRP, _LANES), jnp.float32),
            pltpu.VMEM((_NGENRE, _EMBED), jnp.float32),
            pltpu.VMEM((_EMBED, _BPW), jnp.float32),
            pltpu.SemaphoreType.DMA,
        ],
    )(_mg_body)
    mg = mg_fn(mi, gi, mr, gt)

    up = _tc_expand(ut3, _UTILES)
    ur = up.reshape(2 * _UTILES * 64, _LANES)

    dot_fn = functools.partial(
        pl.kernel,
        mesh=_sc_mesh(),
        compiler_params=_SC_PARAMS,
        out_type=jax.ShapeDtypeStruct((_BATCH,), jnp.float32),
        scratch_types=[
            pltpu.VMEM((_BPW,), jnp.int32),
            pltpu.VMEM((_EMBED, _GRP), jnp.int32),
            pltpu.VMEM((_EMBED, _GRP, _LANES), jnp.float32),
            pltpu.VMEM((_EMBED, _BPW), jnp.float32),
            pltpu.VMEM((_BPW,), jnp.float32),
            pltpu.SemaphoreType.DMA,
        ],
    )(_make_dot_body16())
    return dot_fn(ui, ur, mg)


def kernel(user_indices, movie_indices, genre_indices,
           user_table, movie_table, genre_table):
    ui = user_indices.astype(jnp.int32)
    mi = movie_indices.astype(jnp.int32)
    gi = genre_indices.astype(jnp.int32)
    ut3 = user_table.T.reshape(2, 8, _NUSER)
    mt3 = movie_table.T.reshape(2, 8, _NMOVIE)
    return _run(ui, mi, gi, ut3, mt3, genre_table)
